# baseline (speedup 1.0000x reference)
import functools

import jax
import jax.numpy as jnp
from jax import lax
from jax.experimental import pallas as pl
from jax.experimental.pallas import tpu as pltpu
from jax.experimental.pallas import tpu_sc as plsc

_NC = 2
_NS = 16
_NW = _NC * _NS

_L = 16
_CH = 4096
_VA = 49920
_VB = 50048
_VT = 32


@functools.partial(jax.jit, static_argnums=(3,))
def _sc_row_gather(tab, tail, idx, rows_per_w):
    r_total, v = tab.shape
    f_total, b = idx.shape
    d = r_total // f_total
    n_ch = b // _CH
    mesh = plsc.VectorSubcoreMesh(core_axis_name="c", subcore_axis_name="s")

    @functools.partial(
        pl.kernel,
        out_type=jax.ShapeDtypeStruct((r_total, b), jnp.float32),
        mesh=mesh,
        scratch_types=[
            pltpu.VMEM((_VA,), jnp.float32),
            pltpu.VMEM((_VB,), jnp.float32),
            pltpu.VMEM((rows_per_w * _VT,), jnp.float32),
            pltpu.VMEM((b,), jnp.int32),
            pltpu.VMEM((2, _CH), jnp.float32),
            pltpu.SemaphoreType.DMA,
            pltpu.SemaphoreType.DMA,
            pltpu.SemaphoreType.DMA,
        ],
        compiler_params=pltpu.CompilerParams(needs_layout_passes=False),
    )
    def k(tab_hbm, tail_hbm, idx_hbm, out_hbm,
          a_v, b_v, tails_v, idx_v, out_v, sem_a, sem_b, sem_o):
        wid = lax.axis_index("s") * _NC + lax.axis_index("c")
        row0 = wid * rows_per_w

        pltpu.sync_copy(tail_hbm.at[wid], tails_v)

        def issue_a(rr):
            pltpu.async_copy(tab_hbm.at[rr].at[pl.ds(0, _VA)], a_v, sem_a)

        def issue_b(rr):
            pltpu.async_copy(tab_hbm.at[rr].at[pl.ds(_VA, _VB)], b_v, sem_b)

        def wait_a():
            pltpu.make_async_copy(
                tab_hbm.at[row0].at[pl.ds(0, _VA)], a_v, sem_a
            ).wait()

        def wait_b():
            pltpu.make_async_copy(
                tab_hbm.at[row0].at[pl.ds(_VA, _VB)], b_v, sem_b
            ).wait()

        def drain_out():
            pltpu.make_async_copy(
                out_hbm.at[row0, pl.ds(0, _CH)], out_v.at[0], sem_o
            ).wait()

        issue_a(row0)
        issue_b(row0)

        @pl.loop(0, rows_per_w, init_carry=jnp.int32(-1))
        def _row(kk, prev_f):
            r = row0 + kk
            f = r // d

            @pl.when(f != prev_f)
            def _():
                pltpu.sync_copy(idx_hbm.at[f], idx_v)

            kt_vec = jax.lax.broadcast(kk * _VT, (_L,))

            for c in range(n_ch):
                @pl.when(kk * n_ch + c >= 2)
                def _():
                    drain_out()

                if c == 0:
                    wait_a()

                @plsc.parallel_loop(0, _CH, step=_L, unroll=16)
                def _p1(i):
                    iv = idx_v[pl.ds(c * _CH + i, _L)]
                    ivc = jnp.minimum(iv, _VA - 1)
                    out_v[c % 2, pl.ds(i, _L)] = plsc.load_gather(a_v, [ivc])

                if c == n_ch - 1:
                    @pl.when(kk + 1 < rows_per_w)
                    def _():
                        issue_a(r + 1)

                if c == 0:
                    wait_b()

                @plsc.parallel_loop(0, _CH, step=_L, unroll=8)
                def _p2(i):
                    iv = idx_v[pl.ds(c * _CH + i, _L)]
                    m = iv >= _VA
                    iv2 = jnp.clip(iv - _VA, 0, _VB - 1)
                    vb = plsc.load_gather(b_v, [iv2])
                    mt = iv >= (_VA + _VB)
                    ivt = jnp.clip(iv - (_VA + _VB), 0, _VT - 1)
                    vt = plsc.load_gather(tails_v, [ivt + kt_vec])
                    val = jnp.where(mt, vt, vb)
                    pos = lax.iota(jnp.int32, _L) + i
                    cvec = jax.lax.broadcast(jnp.int32(c % 2), (_L,))
                    plsc.store_scatter(out_v, [cvec, pos], val, mask=m)

                pltpu.async_copy(
                    out_v.at[c % 2], out_hbm.at[r, pl.ds(c * _CH, _CH)], sem_o
                )

                if c == n_ch - 1:
                    @pl.when(kk + 1 < rows_per_w)
                    def _():
                        issue_b(r + 1)

            return f

        drain_out()
        drain_out()

    return k(tab, tail, idx)


def kernel(input, tables):
    f, v, d = tables.shape
    b = input.shape[0]
    r_total = f * d
    assert r_total % _NW == 0 and b % _CH == 0
    assert v == _VA + _VB + _VT
    tab_rows = tables.transpose(0, 2, 1).reshape(r_total, v)
    tab_tail = tab_rows[:, _VA + _VB:].reshape(_NW, (r_total // _NW) * _VT)
    idx_t = input.astype(jnp.int32).T
    out = _sc_row_gather(tab_rows, tab_tail, idx_t, r_total // _NW)
    return out.T.reshape(b, r_total)


# speedup vs baseline: 1.0723x; 1.0723x over previous
import functools

import jax
import jax.numpy as jnp
from jax import lax
from jax.experimental import pallas as pl
from jax.experimental.pallas import tpu as pltpu
from jax.experimental.pallas import tpu_sc as plsc

_NC = 2
_NS = 16
_NW = _NC * _NS

_L = 16
_CH = 4096
_VA = 49920
_VB = 50048
_VT = 32


@functools.partial(jax.jit, static_argnums=(3,))
def _sc_row_gather(tab, tail, idx, rows_per_w):
    r_total, v = tab.shape
    f_total, b = idx.shape
    d = r_total // f_total
    n_ch = b // _CH
    mesh = plsc.VectorSubcoreMesh(core_axis_name="c", subcore_axis_name="s")

    @functools.partial(
        pl.kernel,
        out_type=jax.ShapeDtypeStruct((r_total, b), jnp.float32),
        mesh=mesh,
        scratch_types=[
            pltpu.VMEM((_VA,), jnp.float32),
            pltpu.VMEM((_VB + _VT,), jnp.float32),
            pltpu.VMEM((rows_per_w * _VT,), jnp.float32),
            pltpu.VMEM((b,), jnp.int32),
            pltpu.VMEM((2, _CH), jnp.float32),
            pltpu.SemaphoreType.DMA,
            pltpu.SemaphoreType.DMA,
            pltpu.SemaphoreType.DMA,
        ],
        compiler_params=pltpu.CompilerParams(needs_layout_passes=False),
    )
    def k(tab_hbm, tail_hbm, idx_hbm, out_hbm,
          a_v, b_v, tails_v, idx_v, out_v, sem_a, sem_b, sem_o):
        wid = lax.axis_index("s") * _NC + lax.axis_index("c")
        row0 = wid * rows_per_w

        pltpu.sync_copy(tail_hbm.at[wid], tails_v)

        def issue_a(rr):
            pltpu.async_copy(tab_hbm.at[rr].at[pl.ds(0, _VA)], a_v, sem_a)

        def issue_b(rr):
            pltpu.async_copy(
                tab_hbm.at[rr].at[pl.ds(_VA, _VB)], b_v.at[pl.ds(0, _VB)], sem_b
            )

        def wait_a():
            pltpu.make_async_copy(
                tab_hbm.at[row0].at[pl.ds(0, _VA)], a_v, sem_a
            ).wait()

        def wait_b():
            pltpu.make_async_copy(
                tab_hbm.at[row0].at[pl.ds(_VA, _VB)], b_v.at[pl.ds(0, _VB)],
                sem_b,
            ).wait()

        def drain_out():
            pltpu.make_async_copy(
                out_hbm.at[row0, pl.ds(0, _CH)], out_v.at[0], sem_o
            ).wait()

        issue_a(row0)
        issue_b(row0)

        @pl.loop(0, rows_per_w, init_carry=jnp.int32(-1))
        def _row(kk, prev_f):
            r = row0 + kk
            f = r // d

            @pl.when(f != prev_f)
            def _():
                pltpu.sync_copy(idx_hbm.at[f], idx_v)

            for t in range(_VT // _L):
                b_v[pl.ds(_VB + t * _L, _L)] = tails_v[
                    pl.ds(kk * _VT + t * _L, _L)
                ]

            for c in range(n_ch):
                @pl.when(kk * n_ch + c >= 2)
                def _():
                    drain_out()

                if c == 0:
                    wait_a()

                @plsc.parallel_loop(0, _CH, step=_L, unroll=16)
                def _p1(i):
                    iv = idx_v[pl.ds(c * _CH + i, _L)]
                    ivc = jnp.minimum(iv, _VA - 1)
                    out_v[c % 2, pl.ds(i, _L)] = plsc.load_gather(a_v, [ivc])

                if c == n_ch - 1:
                    @pl.when(kk + 1 < rows_per_w)
                    def _():
                        issue_a(r + 1)

                if c == 0:
                    wait_b()

                @plsc.parallel_loop(0, _CH, step=_L, unroll=16)
                def _p2(i):
                    iv = idx_v[pl.ds(c * _CH + i, _L)]
                    m = iv >= _VA
                    iv2 = jnp.maximum(iv - _VA, 0)
                    vb = plsc.load_gather(b_v, [iv2])
                    pos = lax.iota(jnp.int32, _L) + i
                    cvec = jax.lax.broadcast(jnp.int32(c % 2), (_L,))
                    plsc.store_scatter(out_v, [cvec, pos], vb, mask=m)

                pltpu.async_copy(
                    out_v.at[c % 2], out_hbm.at[r, pl.ds(c * _CH, _CH)], sem_o
                )

                if c == n_ch - 1:
                    @pl.when(kk + 1 < rows_per_w)
                    def _():
                        issue_b(r + 1)

            return f

        drain_out()
        drain_out()

    return k(tab, tail, idx)


def kernel(input, tables):
    f, v, d = tables.shape
    b = input.shape[0]
    r_total = f * d
    assert r_total % _NW == 0 and b % _CH == 0
    assert v == _VA + _VB + _VT
    tab_rows = tables.transpose(0, 2, 1).reshape(r_total, v)
    tab_tail = tab_rows[:, _VA + _VB:].reshape(_NW, (r_total // _NW) * _VT)
    idx_t = input.astype(jnp.int32).T
    out = _sc_row_gather(tab_rows, tab_tail, idx_t, r_total // _NW)
    return out.T.reshape(b, r_total)


# speedup vs baseline: 1.4526x; 1.3547x over previous
import functools

import jax
import jax.numpy as jnp
from jax import lax
from jax.experimental import pallas as pl
from jax.experimental.pallas import tpu as pltpu
from jax.experimental.pallas import tpu_sc as plsc

_NC = 2
_NS = 16
_NW = _NC * _NS

_L = 16
_CH = 4096


@functools.partial(jax.jit, static_argnums=(2,))
def _sc_row_gather(tab, idx, rows_per_w):
    r_total, v = tab.shape
    f_total, b = idx.shape
    d = r_total // f_total
    n_ch = b // _CH
    mesh = plsc.VectorSubcoreMesh(core_axis_name="c", subcore_axis_name="s")

    @functools.partial(
        pl.kernel,
        out_type=jax.ShapeDtypeStruct((r_total, b), jnp.float32),
        mesh=mesh,
        scratch_types=[
            pltpu.VMEM((v,), jnp.float32),
            pltpu.VMEM((b,), jnp.int32),
            pltpu.VMEM((2, _CH), jnp.float32),
            pltpu.SemaphoreType.DMA,
            pltpu.SemaphoreType.DMA,
        ],
        compiler_params=pltpu.CompilerParams(needs_layout_passes=False),
    )
    def k(tab_hbm, idx_hbm, out_hbm, row_v, idx_v, out_v, sem_o, sem_r):
        wid = lax.axis_index("s") * _NC + lax.axis_index("c")
        row0 = wid * rows_per_w

        def drain_out():
            pltpu.make_async_copy(
                out_hbm.at[row0, pl.ds(0, _CH)], out_v.at[0], sem_o
            ).wait()

        @pl.loop(0, rows_per_w, init_carry=jnp.int32(-1))
        def _row(kk, prev_f):
            r = row0 + kk
            f = r // d

            @pl.when(f != prev_f)
            def _():
                pltpu.sync_copy(idx_hbm.at[f], idx_v)

            pltpu.sync_copy(tab_hbm.at[r], row_v)

            for c in range(n_ch):
                @pl.when(kk * n_ch + c >= 2)
                def _():
                    drain_out()

                @plsc.parallel_loop(0, _CH, step=_L, unroll=16)
                def _vec(i):
                    iv = idx_v[pl.ds(c * _CH + i, _L)]
                    out_v[c % 2, pl.ds(i, _L)] = plsc.load_gather(row_v, [iv])

                pltpu.async_copy(
                    out_v.at[c % 2], out_hbm.at[r, pl.ds(c * _CH, _CH)], sem_o
                )
            return f

        drain_out()
        drain_out()

    return k(tab, idx)


def kernel(input, tables):
    f, v, d = tables.shape
    b = input.shape[0]
    r_total = f * d
    assert r_total % _NW == 0 and b % _CH == 0
    tab_rows = tables.transpose(0, 2, 1).reshape(r_total, v)
    idx_t = input.astype(jnp.int32).T
    out = _sc_row_gather(tab_rows, idx_t, r_total // _NW)
    return out.T.reshape(b, r_total)


# speedup vs baseline: 1.4533x; 1.0005x over previous
import functools

import jax
import jax.numpy as jnp
from jax import lax
from jax.experimental import pallas as pl
from jax.experimental.pallas import tpu as pltpu
from jax.experimental.pallas import tpu_sc as plsc

_NC = 2
_NS = 16
_NW = _NC * _NS

_L = 16
_CH = 4096


@functools.partial(jax.jit, static_argnums=(2,))
def _sc_row_gather(tab, idx, rows_per_w):
    r_total, v = tab.shape
    f_total, b = idx.shape
    d = r_total // f_total
    n_ch = b // _CH
    mesh = plsc.VectorSubcoreMesh(core_axis_name="c", subcore_axis_name="s")

    @functools.partial(
        pl.kernel,
        out_type=jax.ShapeDtypeStruct((r_total, b), jnp.float32),
        mesh=mesh,
        scratch_types=[
            pltpu.VMEM((v,), jnp.float32),
            pltpu.VMEM((b,), jnp.int32),
            pltpu.VMEM((2, _CH), jnp.float32),
            pltpu.SemaphoreType.DMA,
            pltpu.SemaphoreType.DMA,
        ],
        compiler_params=pltpu.CompilerParams(needs_layout_passes=False),
    )
    def k(tab_hbm, idx_hbm, out_hbm, row_v, idx_v, out_v, sem_o, sem_r):
        wid = lax.axis_index("s") * _NC + lax.axis_index("c")
        row0 = wid * rows_per_w

        def drain_out():
            pltpu.make_async_copy(
                out_hbm.at[row0, pl.ds(0, _CH)], out_v.at[0], sem_o
            ).wait()

        @pl.loop(0, rows_per_w, init_carry=jnp.int32(-1))
        def _row(kk, prev_f):
            r = row0 + kk
            f = r // d

            @pl.when(f != prev_f)
            def _():
                pltpu.sync_copy(idx_hbm.at[f], idx_v)

            pltpu.sync_copy(tab_hbm.at[r], row_v)

            for c in range(n_ch):
                @pl.when(kk * n_ch + c >= 2)
                def _():
                    drain_out()

                @plsc.parallel_loop(0, _CH, step=_L, unroll=32)
                def _vec(i):
                    iv = idx_v[pl.ds(c * _CH + i, _L)]
                    out_v[c % 2, pl.ds(i, _L)] = plsc.load_gather(row_v, [iv])

                pltpu.async_copy(
                    out_v.at[c % 2], out_hbm.at[r, pl.ds(c * _CH, _CH)], sem_o
                )
            return f

        drain_out()
        drain_out()

    return k(tab, idx)


def kernel(input, tables):
    f, v, d = tables.shape
    b = input.shape[0]
    r_total = f * d
    assert r_total % _NW == 0 and b % _CH == 0
    tab_rows = tables.transpose(0, 2, 1).reshape(r_total, v)
    idx_t = input.astype(jnp.int32).T
    out = _sc_row_gather(tab_rows, idx_t, r_total // _NW)
    return out.T.reshape(b, r_total)
